# Initial kernel scaffold; baseline (speedup 1.0000x reference)
import jax
import jax.numpy as jnp
from jax.experimental import pallas as pl


def kernel(features, edge_index, W, b, gamma, beta):
    raise NotImplementedError("write your pallas kernel here")


# speedup vs baseline: 3.7002x; 3.7002x over previous
import functools

import jax
import jax.numpy as jnp
from jax import lax
from jax.experimental import pallas as pl
from jax.experimental.pallas import tpu as pltpu
from jax.experimental.pallas import tpu_sc as plsc

N = 10000
E = 320000
D = 128
EPS = 1e-5

NC = 2
NS = 16
P = 2
NP_ = 10240
SEG = NP_ // (NC * P)
RPT3 = SEG // NS
CHUNK = 125
EPT1 = E // (NC * NS)
ST1 = EPT1 // 16
CPT3 = (E // NS) // CHUNK
HR = NP_ // 128

_mesh = plsc.VectorSubcoreMesh(
    core_axis_name="c", subcore_axis_name="s", num_cores=NC, num_subcores=NS)


@functools.partial(
    pl.kernel,
    out_type=(
        jax.ShapeDtypeStruct((NC * NS, HR, 128), jnp.int32),
        jax.ShapeDtypeStruct((NC * NS, HR, 128), jnp.int32),
    ),
    mesh=_mesh,
    scratch_types=[
        pltpu.VMEM((EPT1,), jnp.int32),
        pltpu.VMEM((EPT1,), jnp.int32),
        pltpu.VMEM((HR, 128), jnp.int32),
        pltpu.VMEM((HR, 128), jnp.int32),
    ],
    compiler_params=pltpu.CompilerParams(needs_layout_passes=False),
)
def _deg_kernel(src_hbm, dst_hbm, outs_hbm, outd_hbm, sv, dv, degs, degd):
    c = lax.axis_index("c")
    s = lax.axis_index("s")
    wid = c * NS + s
    pltpu.sync_copy(src_hbm.at[pl.ds(wid * EPT1, EPT1)], sv)
    pltpu.sync_copy(dst_hbm.at[pl.ds(wid * EPT1, EPT1)], dv)
    zeros = jnp.zeros((16,), jnp.int32)

    def zero_row(r, carry):
        for k in range(8):
            degs[r, pl.ds(k * 16, 16)] = zeros
            degd[r, pl.ds(k * 16, 16)] = zeros
        return carry

    lax.fori_loop(0, HR, zero_row, 0)

    def step(i, carry):
        v = sv[pl.ds(i * 16, 16)]
        cnt, lm = plsc.scan_count(v)
        plsc.addupdate_scatter(degs, [v >> 7, v & 127], cnt, mask=lm)
        w = dv[pl.ds(i * 16, 16)]
        cnt2, lm2 = plsc.scan_count(w)
        plsc.addupdate_scatter(degd, [w >> 7, w & 127], cnt2, mask=lm2)
        return carry

    lax.fori_loop(0, ST1, step, 0)
    pltpu.sync_copy(degs, outs_hbm.at[wid])
    pltpu.sync_copy(degd, outd_hbm.at[wid])


@functools.partial(
    pl.kernel,
    out_type=jax.ShapeDtypeStruct((NC, P, SEG, D), jnp.float32),
    mesh=_mesh,
    scratch_types=[
        pltpu.VMEM((CPT3, CHUNK), jnp.int32),
        pltpu.VMEM((CPT3, CHUNK), jnp.int32),
        pltpu.VMEM((CHUNK, D), jnp.float32),
        pltpu.VMEM((RPT3, D), jnp.float32),
        pltpu.VMEM_SHARED((SEG, D), jnp.float32),
    ],
)
def _agg_kernel(h_hbm, src_hbm, dst_hbm, zeros_hbm, out_hbm,
                sv, dv, rows_v, zv, agg_sh):
    c = lax.axis_index("c")
    s = lax.axis_index("s")
    row0 = s * RPT3
    for p in range(P):
        pltpu.sync_copy(zeros_hbm, zv)
        pltpu.sync_copy(zv, agg_sh.at[pl.ds(row0, RPT3)])
        pltpu.sync_copy(src_hbm.at[c, p, s], sv)
        pltpu.sync_copy(dst_hbm.at[c, p, s], dv)
        plsc.subcore_barrier()

        def body(j, carry):
            pltpu.sync_copy(
                h_hbm.at[plsc.Indices(sv.at[j], ignored_value=-1)], rows_v)
            pltpu.sync_copy(
                rows_v, agg_sh.at[plsc.Indices(dv.at[j], ignored_value=-1)],
                add=True)
            return carry

        lax.fori_loop(0, CPT3, body, 0)
        plsc.subcore_barrier()
        pltpu.sync_copy(agg_sh.at[pl.ds(row0, RPT3)], zv)
        pltpu.sync_copy(zv, out_hbm.at[c, p, pl.ds(row0, RPT3)])


def _scale_body(f_ref, dsp_ref, ddp_ref, h_ref, nd_ref):
    ds_ = jnp.sum(dsp_ref[...], axis=0).astype(jnp.float32)
    dd_ = jnp.sum(ddp_ref[...], axis=0).astype(jnp.float32)
    ns_ = jnp.where(ds_ > 0, lax.rsqrt(ds_), 0.0)
    nd_ref[...] = jnp.where(dd_ > 0, lax.rsqrt(dd_), 0.0)
    f3 = f_ref[...].reshape(8, 128, 128)
    h_ref[...] = (f3 * ns_[..., None]).reshape(1024, D)


def _final_body(agg_ref, nd_ref, f_ref, w_ref, b_ref, g_ref, be_ref, o_ref):
    a3 = agg_ref[...].reshape(8, 128, 128) * nd_ref[...][..., None]
    a = a3.reshape(1024, D)
    y = jnp.dot(a, w_ref[...], preferred_element_type=jnp.float32) + b_ref[...]
    mean = jnp.mean(y, axis=-1, keepdims=True)
    var = jnp.mean((y - mean) ** 2, axis=-1, keepdims=True)
    y = (y - mean) * lax.rsqrt(var + EPS) * g_ref[...] + be_ref[...]
    o_ref[...] = jnp.maximum(y, 0.0) + f_ref[...]


_BN = 1024


def kernel(features, edge_index, W, b, gamma, beta):
    src = edge_index[0]
    dst = edge_index[1]
    fpad = jnp.pad(features, ((0, NP_ - N), (0, 0)))

    dsp, ddp = _deg_kernel(src, dst)

    h, normd = pl.pallas_call(
        _scale_body,
        grid=(NP_ // _BN,),
        in_specs=[
            pl.BlockSpec((_BN, D), lambda i: (i, 0)),
            pl.BlockSpec((NC * NS, 8, 128), lambda i: (0, i, 0)),
            pl.BlockSpec((NC * NS, 8, 128), lambda i: (0, i, 0)),
        ],
        out_specs=(
            pl.BlockSpec((_BN, D), lambda i: (i, 0)),
            pl.BlockSpec((8, 128), lambda i: (i, 0)),
        ),
        out_shape=(
            jax.ShapeDtypeStruct((NP_, D), jnp.float32),
            jax.ShapeDtypeStruct((HR, 128), jnp.float32),
        ),
    )(fpad, dsp, ddp)

    src_r = src.reshape(NS, CPT3, CHUNK)
    dst_r = dst.reshape(NS, CPT3, CHUNK)
    sg, dsh = [], []
    for q in range(NC * P):
        live = (dst_r >= q * SEG) & (dst_r < (q + 1) * SEG)
        sg.append(jnp.where(live, src_r, -1))
        dsh.append(jnp.where(live, dst_r - q * SEG, -1))
    srcg = jnp.stack(sg).reshape(NC, P, NS, CPT3, CHUNK)
    dsts = jnp.stack(dsh).reshape(NC, P, NS, CPT3, CHUNK)
    zeros160 = jnp.zeros((RPT3, D), jnp.float32)

    aggseg = _agg_kernel(h, srcg, dsts, zeros160)
    agg = aggseg.reshape(NP_, D)

    outp = pl.pallas_call(
        _final_body,
        grid=(NP_ // _BN,),
        in_specs=[
            pl.BlockSpec((_BN, D), lambda i: (i, 0)),
            pl.BlockSpec((8, 128), lambda i: (i, 0)),
            pl.BlockSpec((_BN, D), lambda i: (i, 0)),
            pl.BlockSpec((D, D), lambda i: (0, 0)),
            pl.BlockSpec((1, D), lambda i: (0, 0)),
            pl.BlockSpec((1, D), lambda i: (0, 0)),
            pl.BlockSpec((1, D), lambda i: (0, 0)),
        ],
        out_specs=pl.BlockSpec((_BN, D), lambda i: (i, 0)),
        out_shape=jax.ShapeDtypeStruct((NP_, D), jnp.float32),
    )(agg, normd, fpad, W, b.reshape(1, D), gamma.reshape(1, D),
      beta.reshape(1, D))
    return outp[:N]


# speedup vs baseline: 5.1185x; 1.3833x over previous
import functools

import jax
import jax.numpy as jnp
from jax import lax
from jax.experimental import pallas as pl
from jax.experimental.pallas import tpu as pltpu
from jax.experimental.pallas import tpu_sc as plsc

N = 10000
E = 320000
D = 128
EPS = 1e-5

NC = 2
NS = 16
P = 2
NP_ = 10240
SEG = NP_ // (NC * P)
RPT3 = SEG // NS
CHUNK = 128
EPT1 = E // (NC * NS)
ST1 = EPT1 // 16
CPT3 = 158
NB = 2
HR = NP_ // 128

_mesh = plsc.VectorSubcoreMesh(
    core_axis_name="c", subcore_axis_name="s", num_cores=NC, num_subcores=NS)


@functools.partial(
    pl.kernel,
    out_type=(
        jax.ShapeDtypeStruct((NC * NS, HR, 128), jnp.int32),
        jax.ShapeDtypeStruct((NC * NS, HR, 128), jnp.int32),
    ),
    mesh=_mesh,
    scratch_types=[
        pltpu.VMEM((EPT1,), jnp.int32),
        pltpu.VMEM((EPT1,), jnp.int32),
        pltpu.VMEM((HR, 128), jnp.int32),
        pltpu.VMEM((HR, 128), jnp.int32),
    ],
    compiler_params=pltpu.CompilerParams(needs_layout_passes=False),
)
def _deg_kernel(src_hbm, dst_hbm, outs_hbm, outd_hbm, sv, dv, degs, degd):
    c = lax.axis_index("c")
    s = lax.axis_index("s")
    wid = c * NS + s
    pltpu.sync_copy(src_hbm.at[pl.ds(wid * EPT1, EPT1)], sv)
    pltpu.sync_copy(dst_hbm.at[pl.ds(wid * EPT1, EPT1)], dv)
    zeros = jnp.zeros((16,), jnp.int32)

    def zero_row(r, carry):
        for k in range(8):
            degs[r, pl.ds(k * 16, 16)] = zeros
            degd[r, pl.ds(k * 16, 16)] = zeros
        return carry

    lax.fori_loop(0, HR, zero_row, 0)

    def step(i, carry):
        v = sv[pl.ds(i * 16, 16)]
        cnt, lm = plsc.scan_count(v)
        plsc.addupdate_scatter(degs, [v >> 7, v & 127], cnt, mask=lm)
        w = dv[pl.ds(i * 16, 16)]
        cnt2, lm2 = plsc.scan_count(w)
        plsc.addupdate_scatter(degd, [w >> 7, w & 127], cnt2, mask=lm2)
        return carry

    lax.fori_loop(0, ST1, step, 0)
    pltpu.sync_copy(degs, outs_hbm.at[wid])
    pltpu.sync_copy(degd, outd_hbm.at[wid])


@functools.partial(
    pl.kernel,
    out_type=jax.ShapeDtypeStruct((NC, P, SEG, D), jnp.float32),
    mesh=_mesh,
    scratch_types=[
        pltpu.VMEM((CPT3, CHUNK), jnp.int32),
        pltpu.VMEM((CPT3, CHUNK), jnp.int32),
        [pltpu.VMEM((RPT3, D), jnp.float32) for _ in range(NB)],
        [pltpu.SemaphoreType.DMA for _ in range(NB)],
        [pltpu.SemaphoreType.DMA for _ in range(NB)],
        pltpu.VMEM_SHARED((SEG, D), jnp.float32),
    ],
)
def _agg_kernel(h_hbm, src_hbm, dst_hbm, zeros_hbm, out_hbm,
                sv, dv, rows, gsems, ssems, agg_sh):
    c = lax.axis_index("c")
    s = lax.axis_index("s")
    row0 = s * RPT3
    for p in range(P):
        pltpu.sync_copy(zeros_hbm, rows[0])
        pltpu.sync_copy(rows[0], agg_sh.at[pl.ds(row0, RPT3)])
        pltpu.sync_copy(src_hbm.at[c, p, s], sv)
        pltpu.sync_copy(dst_hbm.at[c, p, s], dv)
        plsc.subcore_barrier()

        def body(jj, carry):
            j0 = jj * NB
            gs = [
                pltpu.async_copy(
                    h_hbm.at[plsc.Indices(sv.at[j0 + b], ignored_value=-1)],
                    rows[b].at[pl.ds(0, CHUNK)], gsems[b])
                for b in range(NB)
            ]
            ss = []
            for b in range(NB):
                gs[b].wait()
                ss.append(pltpu.async_copy(
                    rows[b].at[pl.ds(0, CHUNK)],
                    agg_sh.at[plsc.Indices(dv.at[j0 + b], ignored_value=-1)],
                    ssems[b], add=True))
            for b in range(NB):
                ss[b].wait()
            return carry

        lax.fori_loop(0, CPT3 // NB, body, 0)
        plsc.subcore_barrier()
        pltpu.sync_copy(agg_sh.at[pl.ds(row0, RPT3)], rows[0])
        pltpu.sync_copy(rows[0], out_hbm.at[c, p, pl.ds(row0, RPT3)])


def _scale_body(f_ref, dsp_ref, ddp_ref, h_ref, nd_ref):
    ds_ = jnp.sum(dsp_ref[...], axis=0).astype(jnp.float32)
    dd_ = jnp.sum(ddp_ref[...], axis=0).astype(jnp.float32)
    ns_ = jnp.where(ds_ > 0, lax.rsqrt(ds_), 0.0)
    nd_ref[...] = jnp.where(dd_ > 0, lax.rsqrt(dd_), 0.0)
    f3 = f_ref[...].reshape(8, 128, 128)
    h_ref[...] = (f3 * ns_[..., None]).reshape(1024, D)


def _final_body(agg_ref, nd_ref, f_ref, w_ref, b_ref, g_ref, be_ref, o_ref):
    a3 = agg_ref[...].reshape(8, 128, 128) * nd_ref[...][..., None]
    a = a3.reshape(1024, D)
    y = jnp.dot(a, w_ref[...], preferred_element_type=jnp.float32) + b_ref[...]
    mean = jnp.mean(y, axis=-1, keepdims=True)
    var = jnp.mean((y - mean) ** 2, axis=-1, keepdims=True)
    y = (y - mean) * lax.rsqrt(var + EPS) * g_ref[...] + be_ref[...]
    o_ref[...] = jnp.maximum(y, 0.0) + f_ref[...]


_BN = 1024


def kernel(features, edge_index, W, b, gamma, beta):
    src = edge_index[0]
    dst = edge_index[1]
    fpad = jnp.pad(features, ((0, NP_ - N), (0, 0)))

    dsp, ddp = _deg_kernel(src, dst)

    h, normd = pl.pallas_call(
        _scale_body,
        grid=(NP_ // _BN,),
        in_specs=[
            pl.BlockSpec((_BN, D), lambda i: (i, 0)),
            pl.BlockSpec((NC * NS, 8, 128), lambda i: (0, i, 0)),
            pl.BlockSpec((NC * NS, 8, 128), lambda i: (0, i, 0)),
        ],
        out_specs=(
            pl.BlockSpec((_BN, D), lambda i: (i, 0)),
            pl.BlockSpec((8, 128), lambda i: (i, 0)),
        ),
        out_shape=(
            jax.ShapeDtypeStruct((NP_, D), jnp.float32),
            jax.ShapeDtypeStruct((HR, 128), jnp.float32),
        ),
    )(fpad, dsp, ddp)

    pad = jnp.full((NS, CPT3 * CHUNK - E // NS), -1, jnp.int32)
    src_r = jnp.concatenate([src.reshape(NS, E // NS), pad], axis=1)
    src_r = src_r.reshape(NS, CPT3, CHUNK)
    dst_r = jnp.concatenate([dst.reshape(NS, E // NS), pad], axis=1)
    dst_r = dst_r.reshape(NS, CPT3, CHUNK)
    sg, dsh = [], []
    for q in range(NC * P):
        live = (dst_r >= q * SEG) & (dst_r < (q + 1) * SEG)
        sg.append(jnp.where(live, src_r, -1))
        dsh.append(jnp.where(live, dst_r - q * SEG, -1))
    srcg = jnp.stack(sg).reshape(NC, P, NS, CPT3, CHUNK)
    dsts = jnp.stack(dsh).reshape(NC, P, NS, CPT3, CHUNK)
    zeros160 = jnp.zeros((RPT3, D), jnp.float32)

    aggseg = _agg_kernel(h, srcg, dsts, zeros160)
    agg = aggseg.reshape(NP_, D)

    outp = pl.pallas_call(
        _final_body,
        grid=(NP_ // _BN,),
        in_specs=[
            pl.BlockSpec((_BN, D), lambda i: (i, 0)),
            pl.BlockSpec((8, 128), lambda i: (i, 0)),
            pl.BlockSpec((_BN, D), lambda i: (i, 0)),
            pl.BlockSpec((D, D), lambda i: (0, 0)),
            pl.BlockSpec((1, D), lambda i: (0, 0)),
            pl.BlockSpec((1, D), lambda i: (0, 0)),
            pl.BlockSpec((1, D), lambda i: (0, 0)),
        ],
        out_specs=pl.BlockSpec((_BN, D), lambda i: (i, 0)),
        out_shape=jax.ShapeDtypeStruct((NP_, D), jnp.float32),
    )(agg, normd, fpad, W, b.reshape(1, D), gamma.reshape(1, D),
      beta.reshape(1, D))
    return outp[:N]


# speedup vs baseline: 5.7164x; 1.1168x over previous
import functools

import jax
import jax.numpy as jnp
from jax import lax
from jax.experimental import pallas as pl
from jax.experimental.pallas import tpu as pltpu
from jax.experimental.pallas import tpu_sc as plsc

N = 10000
E = 320000
D = 128
EPS = 1e-5

NC = 2
NS = 16
P = 2
NP_ = 10240
SEG = NP_ // (NC * P)
RPT3 = SEG // NS
CHUNK = 128
EPT1 = E // (NC * NS)
ST1 = EPT1 // 16
CPT3 = 160
HF = 2
CPH = CPT3 // HF
NB = 4
HR = NP_ // 128

_mesh = plsc.VectorSubcoreMesh(
    core_axis_name="c", subcore_axis_name="s", num_cores=NC, num_subcores=NS)


@functools.partial(
    pl.kernel,
    out_type=(
        jax.ShapeDtypeStruct((NC * NS, HR, 128), jnp.int32),
        jax.ShapeDtypeStruct((NC * NS, HR, 128), jnp.int32),
    ),
    mesh=_mesh,
    scratch_types=[
        pltpu.VMEM((EPT1,), jnp.int32),
        pltpu.VMEM((EPT1,), jnp.int32),
        pltpu.VMEM((HR, 128), jnp.int32),
        pltpu.VMEM((HR, 128), jnp.int32),
    ],
    compiler_params=pltpu.CompilerParams(needs_layout_passes=False),
)
def _deg_kernel(src_hbm, dst_hbm, outs_hbm, outd_hbm, sv, dv, degs, degd):
    c = lax.axis_index("c")
    s = lax.axis_index("s")
    wid = c * NS + s
    pltpu.sync_copy(src_hbm.at[pl.ds(wid * EPT1, EPT1)], sv)
    pltpu.sync_copy(dst_hbm.at[pl.ds(wid * EPT1, EPT1)], dv)
    zeros = jnp.zeros((16,), jnp.int32)

    def zero_row(r, carry):
        for k in range(8):
            degs[r, pl.ds(k * 16, 16)] = zeros
            degd[r, pl.ds(k * 16, 16)] = zeros
        return carry

    lax.fori_loop(0, HR, zero_row, 0)

    def step(i, carry):
        v = sv[pl.ds(i * 16, 16)]
        cnt, lm = plsc.scan_count(v)
        plsc.addupdate_scatter(degs, [v >> 7, v & 127], cnt, mask=lm)
        w = dv[pl.ds(i * 16, 16)]
        cnt2, lm2 = plsc.scan_count(w)
        plsc.addupdate_scatter(degd, [w >> 7, w & 127], cnt2, mask=lm2)
        return carry

    lax.fori_loop(0, ST1, step, 0)
    pltpu.sync_copy(degs, outs_hbm.at[wid])
    pltpu.sync_copy(degd, outd_hbm.at[wid])


@functools.partial(
    pl.kernel,
    out_type=jax.ShapeDtypeStruct((NC, P, SEG, D), jnp.float32),
    mesh=_mesh,
    scratch_types=[
        pltpu.VMEM((CPH, CHUNK), jnp.int32),
        pltpu.VMEM((CPH, CHUNK), jnp.int32),
        [pltpu.VMEM((RPT3, D), jnp.float32) for _ in range(NB)],
        [pltpu.SemaphoreType.DMA for _ in range(NB)],
        [pltpu.SemaphoreType.DMA for _ in range(NB)],
        pltpu.VMEM_SHARED((SEG, D), jnp.float32),
    ],
)
def _agg_kernel(h_hbm, src_hbm, dst_hbm, zeros_hbm, out_hbm,
                sv, dv, rows, gsems, ssems, agg_sh):
    c = lax.axis_index("c")
    s = lax.axis_index("s")
    row0 = s * RPT3
    for p in range(P):
        pltpu.sync_copy(zeros_hbm, rows[0])
        pltpu.sync_copy(rows[0], agg_sh.at[pl.ds(row0, RPT3)])
        plsc.subcore_barrier()

        for hf in range(HF):
            pltpu.sync_copy(src_hbm.at[c, p, s, hf], sv)
            pltpu.sync_copy(dst_hbm.at[c, p, s, hf], dv)

            def body(jj, carry):
                j0 = jj * NB
                gs = [
                    pltpu.async_copy(
                        h_hbm.at[plsc.Indices(sv.at[j0 + b], ignored_value=-1)],
                        rows[b].at[pl.ds(0, CHUNK)], gsems[b])
                    for b in range(NB)
                ]
                ss = []
                for b in range(NB):
                    gs[b].wait()
                    ss.append(pltpu.async_copy(
                        rows[b].at[pl.ds(0, CHUNK)],
                        agg_sh.at[plsc.Indices(dv.at[j0 + b], ignored_value=-1)],
                        ssems[b], add=True))
                for b in range(NB):
                    ss[b].wait()
                return carry

            lax.fori_loop(0, CPH // NB, body, 0)
        plsc.subcore_barrier()
        pltpu.sync_copy(agg_sh.at[pl.ds(row0, RPT3)], rows[0])
        pltpu.sync_copy(rows[0], out_hbm.at[c, p, pl.ds(row0, RPT3)])


def _scale_body(f_ref, dsp_ref, ddp_ref, h_ref, nd_ref):
    ds_ = jnp.sum(dsp_ref[...], axis=0).astype(jnp.float32)
    dd_ = jnp.sum(ddp_ref[...], axis=0).astype(jnp.float32)
    ns_ = jnp.where(ds_ > 0, lax.rsqrt(ds_), 0.0)
    nd_ref[...] = jnp.where(dd_ > 0, lax.rsqrt(dd_), 0.0)
    f3 = f_ref[...].reshape(8, 128, 128)
    h_ref[...] = (f3 * ns_[..., None]).reshape(1024, D)


def _final_body(agg_ref, nd_ref, f_ref, w_ref, b_ref, g_ref, be_ref, o_ref):
    a3 = agg_ref[...].reshape(8, 128, 128) * nd_ref[...][..., None]
    a = a3.reshape(1024, D)
    y = jnp.dot(a, w_ref[...], preferred_element_type=jnp.float32) + b_ref[...]
    mean = jnp.mean(y, axis=-1, keepdims=True)
    var = jnp.mean((y - mean) ** 2, axis=-1, keepdims=True)
    y = (y - mean) * lax.rsqrt(var + EPS) * g_ref[...] + be_ref[...]
    o_ref[...] = jnp.maximum(y, 0.0) + f_ref[...]


_BN = 1024


def kernel(features, edge_index, W, b, gamma, beta):
    src = edge_index[0]
    dst = edge_index[1]
    fpad = jnp.pad(features, ((0, NP_ - N), (0, 0)))

    dsp, ddp = _deg_kernel(src, dst)

    h, normd = pl.pallas_call(
        _scale_body,
        grid=(NP_ // _BN,),
        in_specs=[
            pl.BlockSpec((_BN, D), lambda i: (i, 0)),
            pl.BlockSpec((NC * NS, 8, 128), lambda i: (0, i, 0)),
            pl.BlockSpec((NC * NS, 8, 128), lambda i: (0, i, 0)),
        ],
        out_specs=(
            pl.BlockSpec((_BN, D), lambda i: (i, 0)),
            pl.BlockSpec((8, 128), lambda i: (i, 0)),
        ),
        out_shape=(
            jax.ShapeDtypeStruct((NP_, D), jnp.float32),
            jax.ShapeDtypeStruct((HR, 128), jnp.float32),
        ),
    )(fpad, dsp, ddp)

    pad = jnp.full((NS, CPT3 * CHUNK - E // NS), -1, jnp.int32)
    src_r = jnp.concatenate([src.reshape(NS, E // NS), pad], axis=1)
    src_r = src_r.reshape(NS, CPT3, CHUNK)
    dst_r = jnp.concatenate([dst.reshape(NS, E // NS), pad], axis=1)
    dst_r = dst_r.reshape(NS, CPT3, CHUNK)
    sg, dsh = [], []
    for q in range(NC * P):
        live = (dst_r >= q * SEG) & (dst_r < (q + 1) * SEG)
        sg.append(jnp.where(live, src_r, -1))
        dsh.append(jnp.where(live, dst_r - q * SEG, -1))
    srcg = jnp.stack(sg).reshape(NC, P, NS, HF, CPH, CHUNK)
    dsts = jnp.stack(dsh).reshape(NC, P, NS, HF, CPH, CHUNK)
    zeros160 = jnp.zeros((RPT3, D), jnp.float32)

    aggseg = _agg_kernel(h, srcg, dsts, zeros160)
    agg = aggseg.reshape(NP_, D)

    outp = pl.pallas_call(
        _final_body,
        grid=(NP_ // _BN,),
        in_specs=[
            pl.BlockSpec((_BN, D), lambda i: (i, 0)),
            pl.BlockSpec((8, 128), lambda i: (i, 0)),
            pl.BlockSpec((_BN, D), lambda i: (i, 0)),
            pl.BlockSpec((D, D), lambda i: (0, 0)),
            pl.BlockSpec((1, D), lambda i: (0, 0)),
            pl.BlockSpec((1, D), lambda i: (0, 0)),
            pl.BlockSpec((1, D), lambda i: (0, 0)),
        ],
        out_specs=pl.BlockSpec((_BN, D), lambda i: (i, 0)),
        out_shape=jax.ShapeDtypeStruct((NP_, D), jnp.float32),
    )(agg, normd, fpad, W, b.reshape(1, D), gamma.reshape(1, D),
      beta.reshape(1, D))
    return outp[:N]
